# Initial kernel scaffold; baseline (speedup 1.0000x reference)
import jax
import jax.numpy as jnp
from jax.experimental import pallas as pl


def kernel(x, W_enc_leaf, b_enc_leaf, W_enc_mid, b_enc_mid, w_dec_mid, b_dec_mid, w_dec_leaf, b_dec_leaf, w_dec_gene, b_dec_gene):
    raise NotImplementedError("write your pallas kernel here")


# speedup vs baseline: 2.2883x; 2.2883x over previous
import functools

import jax
import jax.numpy as jnp
import numpy as np
from jax.experimental import pallas as pl
from jax.experimental.pallas import tpu as pltpu

_B = 2048
_G = 2048
_N_LEAF = 256
_GPL = 8
_N_MID = 64
_LPM = 4
_EPS = 1e-5
_BLK_G = 128
_BLK_LEAF = _BLK_G // _GPL
_BLK_MID = _BLK_LEAF // _LPM
_NBLK = _G // _BLK_G


def _onto_block_kernel(x_ref, wl_ref, bl_ref, wm_ref, bm_ref,
                       wdl_ref, bdl_ref, wdg_ref, bdg_ref, out_ref):
    xb = x_ref[...]
    mu = jnp.mean(xb, axis=0, keepdims=True)
    var = jnp.mean(xb * xb, axis=0, keepdims=True) - mu * mu
    xn = (xb - mu) * jax.lax.rsqrt(var + _EPS)
    hp = jnp.dot(xn, wl_ref[0], preferred_element_type=jnp.float32)
    h = jnp.maximum(hp + bl_ref[0], 0.0)
    muh = jnp.mean(h, axis=0, keepdims=True)
    varh = jnp.mean(h * h, axis=0, keepdims=True) - muh * muh
    hn = (h - muh) * jax.lax.rsqrt(varh + _EPS)
    zp = jnp.dot(hn, wm_ref[0], preferred_element_type=jnp.float32)
    z = jnp.maximum(zp + bm_ref[0], 0.0)
    e4 = (jax.lax.broadcasted_iota(jnp.int32, (_BLK_MID, _BLK_LEAF), 1)
          // _LPM == jax.lax.broadcasted_iota(
              jnp.int32, (_BLK_MID, _BLK_LEAF), 0)).astype(jnp.float32)
    zx = jnp.dot(z, e4, preferred_element_type=jnp.float32)
    dl = jnp.maximum(zx * wdl_ref[0] + bdl_ref[0], 0.0)
    e16 = (jax.lax.broadcasted_iota(jnp.int32, (_BLK_LEAF, _BLK_G), 1)
           // _GPL == jax.lax.broadcasted_iota(
               jnp.int32, (_BLK_LEAF, _BLK_G), 0)).astype(jnp.float32)
    dx = jnp.dot(dl, e16, preferred_element_type=jnp.float32)
    out_ref[...] = dx * wdg_ref[0] + bdg_ref[0]


@functools.partial(jax.jit, static_argnames=())
def kernel(x, W_enc_leaf, b_enc_leaf, W_enc_mid, b_enc_mid,
           w_dec_mid, b_dec_mid, w_dec_leaf, b_dec_leaf,
           w_dec_gene, b_dec_gene):
    f32 = jnp.float32
    gl = np.arange(_BLK_G)
    tl = np.arange(_BLK_LEAF)
    leaf_mask = (gl[:, None] // _GPL == tl[None, :])
    w_leaf_b = W_enc_leaf.reshape(_NBLK, _BLK_LEAF, _GPL)
    wl = jnp.where(leaf_mask[None],
                   w_leaf_b.transpose(0, 2, 1)[:, gl % _GPL, :], 0.0)
    bl = b_enc_leaf.reshape(_NBLK, 1, _BLK_LEAF)
    mid_mask = (tl[:, None] // _LPM == np.arange(_BLK_MID)[None, :])
    w_mid_b = W_enc_mid.reshape(_NBLK, _BLK_MID, _LPM)
    wm = jnp.where(mid_mask[None],
                   w_mid_b.transpose(0, 2, 1)[:, tl % _LPM, :], 0.0)
    bm = b_enc_mid.reshape(_NBLK, 1, _BLK_MID)
    wdl = w_dec_leaf.reshape(_NBLK, 1, _BLK_LEAF)
    bdl = b_dec_leaf.reshape(_NBLK, 1, _BLK_LEAF)
    wdg = w_dec_gene.reshape(_NBLK, 1, _BLK_G)
    bdg = b_dec_gene.reshape(_NBLK, 1, _BLK_G)

    grid = (_NBLK,)
    out = pl.pallas_call(
        _onto_block_kernel,
        grid=grid,
        in_specs=[
            pl.BlockSpec((_B, _BLK_G), lambda j: (0, j)),
            pl.BlockSpec((1, _BLK_G, _BLK_LEAF), lambda j: (j, 0, 0)),
            pl.BlockSpec((1, 1, _BLK_LEAF), lambda j: (j, 0, 0)),
            pl.BlockSpec((1, _BLK_LEAF, _BLK_MID), lambda j: (j, 0, 0)),
            pl.BlockSpec((1, 1, _BLK_MID), lambda j: (j, 0, 0)),
            pl.BlockSpec((1, 1, _BLK_LEAF), lambda j: (j, 0, 0)),
            pl.BlockSpec((1, 1, _BLK_LEAF), lambda j: (j, 0, 0)),
            pl.BlockSpec((1, 1, _BLK_G), lambda j: (j, 0, 0)),
            pl.BlockSpec((1, 1, _BLK_G), lambda j: (j, 0, 0)),
        ],
        out_specs=pl.BlockSpec((_B, _BLK_G), lambda j: (0, j)),
        out_shape=jax.ShapeDtypeStruct((_B, _G), f32),
    )(x, wl, bl, wm, bm, wdl, bdl, wdg, bdg)
    return out
